# baseline (speedup 1.0000x reference)
import jax
import jax.numpy as jnp
from jax import lax
from jax.experimental import pallas as pl
from jax.experimental.pallas import tpu as pltpu

_E = 16
_BLK = 1024


def _router_body(x_ref, wt_ref, w_out_ref, i_out_ref):
    logits = jax.lax.dot_general(
        x_ref[...], wt_ref[...],
        dimension_numbers=(((1,), (0,)), ((), ())),
        precision=lax.Precision.DEFAULT,
        preferred_element_type=jnp.float32,
    )
    iota_e = lax.broadcasted_iota(jnp.int32, (_BLK, _E), 1)
    m1 = jnp.max(logits, axis=1, keepdims=True)
    i1 = jnp.min(jnp.where(logits == m1, iota_e, _E), axis=1, keepdims=True)
    masked = jnp.where(iota_e == i1, -jnp.inf, logits)
    m2 = jnp.max(masked, axis=1, keepdims=True)
    i2 = jnp.min(jnp.where(masked == m2, iota_e, _E), axis=1, keepdims=True)
    e2 = jnp.exp(m2 - m1)
    w1 = 1.0 / (1.0 + e2)
    w2 = e2 * w1
    w_out_ref[...] = jnp.transpose(jnp.concatenate([w1, w2], axis=1))
    i_out_ref[...] = jnp.transpose(jnp.concatenate([i1, i2], axis=1))


@jax.jit
def kernel(x, W):
    B, T, D = x.shape
    n_tok = B * T
    xf = x.reshape(n_tok, D)
    wt = W.T

    grid = (n_tok // _BLK,)
    w_out, i_out = pl.pallas_call(
        _router_body,
        grid=grid,
        in_specs=[
            pl.BlockSpec((_BLK, D), lambda i: (i, 0)),
            pl.BlockSpec((D, _E), lambda i: (0, 0)),
        ],
        out_specs=[
            pl.BlockSpec((2, _BLK), lambda i: (0, i)),
            pl.BlockSpec((2, _BLK), lambda i: (0, i)),
        ],
        out_shape=[
            jax.ShapeDtypeStruct((2, n_tok), jnp.float32),
            jax.ShapeDtypeStruct((2, n_tok), jnp.int32),
        ],
        compiler_params=pltpu.CompilerParams(
            dimension_semantics=("arbitrary",),
        ),
    )(xf, wt)

    return (w_out.T.reshape(B, T, 2), i_out.T.reshape(B, T, 2))


# speedup vs baseline: 1.0650x; 1.0650x over previous
import jax
import jax.numpy as jnp
from jax import lax
from jax.experimental import pallas as pl
from jax.experimental.pallas import tpu as pltpu

_E = 16
_BLK = 2048
_DH = 1024


def _router_body(xa_ref, xb_ref, wta_ref, wtb_ref, w_out_ref, i_out_ref):
    logits = jnp.dot(xa_ref[...], wta_ref[...], preferred_element_type=jnp.float32)
    logits = logits + jnp.dot(xb_ref[...], wtb_ref[...], preferred_element_type=jnp.float32)
    iota_e = lax.broadcasted_iota(jnp.int32, (_BLK, _E), 1)
    m1 = jnp.max(logits, axis=1, keepdims=True)
    i1 = jnp.min(jnp.where(logits == m1, iota_e, _E), axis=1, keepdims=True)
    masked = jnp.where(iota_e == i1, -jnp.inf, logits)
    m2 = jnp.max(masked, axis=1, keepdims=True)
    i2 = jnp.min(jnp.where(masked == m2, iota_e, _E), axis=1, keepdims=True)
    e2 = jnp.exp(m2 - m1)
    w1 = 1.0 / (1.0 + e2)
    w2 = e2 * w1
    w_out_ref[...] = jnp.transpose(jnp.concatenate([w1, w2], axis=1))
    i_out_ref[...] = jnp.transpose(jnp.concatenate([i1, i2], axis=1))


@jax.jit
def kernel(x, W):
    B, T, D = x.shape
    n_tok = B * T
    xf = x.reshape(n_tok, D)
    wt = W.T

    grid = (n_tok // _BLK,)
    w_out, i_out = pl.pallas_call(
        _router_body,
        grid=grid,
        in_specs=[
            pl.BlockSpec((_BLK, _DH), lambda i: (i, 0)),
            pl.BlockSpec((_BLK, _DH), lambda i: (i, 1)),
            pl.BlockSpec((_DH, _E), lambda i: (0, 0)),
            pl.BlockSpec((_DH, _E), lambda i: (1, 0)),
        ],
        out_specs=[
            pl.BlockSpec((2, _BLK), lambda i: (0, i)),
            pl.BlockSpec((2, _BLK), lambda i: (0, i)),
        ],
        out_shape=[
            jax.ShapeDtypeStruct((2, n_tok), jnp.float32),
            jax.ShapeDtypeStruct((2, n_tok), jnp.int32),
        ],
        compiler_params=pltpu.CompilerParams(
            dimension_semantics=("arbitrary",),
        ),
    )(xf, xf, wt, wt)

    return (w_out.T.reshape(B, T, 2), i_out.T.reshape(B, T, 2))
